# Initial kernel scaffold; baseline (speedup 1.0000x reference)
import functools

import jax
import jax.numpy as jnp
from jax import lax
from jax.experimental import pallas as pl
from jax.experimental.pallas import tpu as pltpu
from jax.experimental.pallas import tpu_sc as plsc

B = 4096
S = 200
D = 64
NW = 32
B_PER_W = B // NW
HALF = S // 2


def _embedder(idx_hbm, tok_hbm, pos_hbm, out_hbm, idx_v, pos_v, buf, sem):
    wid = lax.axis_index("s") * 2 + lax.axis_index("c")
    pltpu.sync_copy(pos_hbm, pos_v)

    def body(i, carry):
        b = wid * B_PER_W + i
        pltpu.sync_copy(idx_hbm.at[b], idx_v)
        pltpu.sync_copy(pos_v, buf)
        cp0 = pltpu.async_copy(
            tok_hbm.at[idx_v.at[0]], buf.at[pl.ds(0, HALF)], sem, add=True
        )
        cp1 = pltpu.async_copy(
            tok_hbm.at[idx_v.at[1]], buf.at[pl.ds(HALF, HALF)], sem, add=True
        )
        cp0.wait()
        cp1.wait()
        pltpu.sync_copy(buf, out_hbm.at[b])
        return carry

    lax.fori_loop(0, B_PER_W, body, 0)


@jax.jit
def _run(idx, token_table, pos_table):
    kern = pl.kernel(
        _embedder,
        out_type=jax.ShapeDtypeStruct((B, S, D), jnp.float32),
        mesh=plsc.VectorSubcoreMesh(core_axis_name="c", subcore_axis_name="s"),
        scratch_types=[
            pltpu.VMEM((2, HALF), jnp.int32),
            pltpu.VMEM((S, D), jnp.float32),
            pltpu.VMEM((S, D), jnp.float32),
            pltpu.SemaphoreType.DMA,
        ],
    )
    return kern(idx, token_table, pos_table)


def kernel(input, token_table, pos_table):
    idx = input.astype(jnp.int32).reshape(B, 2, HALF)
    return _run(idx, token_table, pos_table)


# speedup vs baseline: 2.3637x; 2.3637x over previous
import functools

import jax
import jax.numpy as jnp
from jax import lax
from jax.experimental import pallas as pl
from jax.experimental.pallas import tpu as pltpu
from jax.experimental.pallas import tpu_sc as plsc

B = 4096
S = 200
D = 64
NW = 32
B_PER_W = B // NW
HALF = S // 2


def _embedder(idx_hbm, tok_hbm, pos_hbm, out_hbm, idx_v, pos_v, buf, sem):
    wid = lax.axis_index("s") * 2 + lax.axis_index("c")
    pltpu.sync_copy(pos_hbm, pos_v)

    def body(i, carry):
        b = wid * B_PER_W + i
        pltpu.sync_copy(idx_hbm.at[b], idx_v)
        cp0 = pltpu.async_copy(tok_hbm.at[idx_v.at[0]], buf.at[pl.ds(0, HALF)], sem)
        cp1 = pltpu.async_copy(tok_hbm.at[idx_v.at[1]], buf.at[pl.ds(HALF, HALF)], sem)
        cp0.wait()
        cp1.wait()

        def add_row(r):
            for c in range(D // 16):
                sl = pl.ds(c * 16, 16)
                plsc.addupdate(buf.at[r, sl], pos_v[r, sl])

        plsc.parallel_loop(0, S, 1, unroll=4)(add_row)
        pltpu.sync_copy(buf, out_hbm.at[b])
        return carry

    lax.fori_loop(0, B_PER_W, body, 0)


@jax.jit
def _run(idx, token_table, pos_table):
    kern = pl.kernel(
        _embedder,
        out_type=jax.ShapeDtypeStruct((B, S, D), jnp.float32),
        mesh=plsc.VectorSubcoreMesh(core_axis_name="c", subcore_axis_name="s"),
        scratch_types=[
            pltpu.VMEM((2, HALF), jnp.int32),
            pltpu.VMEM((S, D), jnp.float32),
            pltpu.VMEM((S, D), jnp.float32),
            pltpu.SemaphoreType.DMA,
        ],
        compiler_params=pltpu.CompilerParams(use_tc_tiling_on_sc=False),
    )
    return kern(idx, token_table, pos_table)


def kernel(input, token_table, pos_table):
    idx = input.astype(jnp.int32).reshape(B, 2, HALF)
    return _run(idx, token_table, pos_table)


# speedup vs baseline: 2.4724x; 1.0460x over previous
import functools

import jax
import jax.numpy as jnp
from jax import lax
from jax.experimental import pallas as pl
from jax.experimental.pallas import tpu as pltpu
from jax.experimental.pallas import tpu_sc as plsc

B = 4096
S = 200
D = 64
NW = 32
B_PER_W = B // NW
HALF = S // 2
NBUF = 4
LOOKAHEAD = 2


def _embedder(idx_hbm, tok_hbm, pos_hbm, out_hbm, idx_v, pos_v, buf, gsem, osem):
    wid = lax.axis_index("s") * 2 + lax.axis_index("c")
    pltpu.sync_copy(pos_hbm, pos_v)
    pltpu.sync_copy(idx_hbm.at[wid], idx_v)

    def start_gather(i, slot):
        for h in range(2):
            pltpu.async_copy(tok_hbm.at[idx_v.at[i, h]], buf.at[slot, h], gsem.at[slot])

    def start_store(i, slot):
        return pltpu.async_copy(buf.at[slot], out_hbm.at[wid, i], osem.at[slot])

    def wait_gather(i, slot):
        for h in range(2):
            pltpu.make_async_copy(
                tok_hbm.at[idx_v.at[i, h]], buf.at[slot, h], gsem.at[slot]
            ).wait()

    def wait_store(i, slot):
        pltpu.make_async_copy(buf.at[slot], out_hbm.at[wid, i], osem.at[slot]).wait()

    for j in range(LOOKAHEAD):
        start_gather(j, j)

    def body(i, carry):
        s = lax.rem(i, NBUF)
        j = i + LOOKAHEAD
        sj = lax.rem(j, NBUF)

        @pl.when(j < B_PER_W)
        def _prefetch():
            @pl.when(j >= NBUF)
            def _drain():
                wait_store(j - NBUF, sj)

            start_gather(j, sj)

        wait_gather(i, s)

        def add_row(r):
            for h in range(2):
                for c in range(D // 16):
                    sl = pl.ds(c * 16, 16)
                    plsc.addupdate(buf.at[s, h, r, sl], pos_v[h, r, sl])

        plsc.parallel_loop(0, HALF, 1, unroll=4)(add_row)
        start_store(i, s)
        return carry

    lax.fori_loop(0, B_PER_W, body, 0)

    for i in range(B_PER_W - NBUF, B_PER_W):
        wait_store(i, i % NBUF)


@jax.jit
def _run(idx, token_table, pos_table):
    kern = pl.kernel(
        _embedder,
        out_type=jax.ShapeDtypeStruct((NW, B_PER_W, 2, HALF, D), jnp.float32),
        mesh=plsc.VectorSubcoreMesh(core_axis_name="c", subcore_axis_name="s"),
        scratch_types=[
            pltpu.VMEM((B_PER_W, 2, HALF), jnp.int32),
            pltpu.VMEM((2, HALF, D), jnp.float32),
            pltpu.VMEM((NBUF, 2, HALF, D), jnp.float32),
            pltpu.SemaphoreType.DMA((NBUF,)),
            pltpu.SemaphoreType.DMA((NBUF,)),
        ],
        compiler_params=pltpu.CompilerParams(use_tc_tiling_on_sc=False),
    )
    return kern(idx, token_table, pos_table)


def kernel(input, token_table, pos_table):
    idx = input.astype(jnp.int32).reshape(NW, B_PER_W, 2, HALF)
    pos = pos_table.reshape(2, HALF, D)
    out = _run(idx, token_table, pos)
    return out.reshape(B, S, D)


# speedup vs baseline: 2.9404x; 1.1893x over previous
import functools

import jax
import jax.numpy as jnp
from jax import lax
from jax.experimental import pallas as pl
from jax.experimental.pallas import tpu as pltpu
from jax.experimental.pallas import tpu_sc as plsc

B = 4096
S = 200
D = 64
NW = 32
BC = B // NW
NBUF = 4
LOOKAHEAD = 2


def _embedder(idx_hbm, tok_hbm, pos_hbm, out_hbm, idx_v, pos_v, buf, gsem, osem):
    wid = lax.axis_index("s") * 2 + lax.axis_index("c")
    b0 = wid * BC
    pltpu.sync_copy(pos_hbm, pos_v)
    pltpu.sync_copy(idx_hbm.at[:, pl.ds(b0, BC)], idx_v)

    def start_gather(i, slot):
        pltpu.async_copy(tok_hbm.at[idx_v.at[i]], buf.at[slot], gsem.at[slot])

    def start_store(i, slot):
        pltpu.async_copy(buf.at[slot], out_hbm.at[i, pl.ds(b0, BC)], osem.at[slot])

    def wait_gather(i, slot):
        pltpu.make_async_copy(tok_hbm.at[idx_v.at[i]], buf.at[slot], gsem.at[slot]).wait()

    def wait_store(i, slot):
        pltpu.make_async_copy(
            buf.at[slot], out_hbm.at[i, pl.ds(b0, BC)], osem.at[slot]
        ).wait()

    for j in range(LOOKAHEAD):
        start_gather(j, j)

    def body(i, carry):
        s = lax.rem(i, NBUF)
        j = i + LOOKAHEAD
        sj = lax.rem(j, NBUF)

        @pl.when(j < S)
        def _prefetch():
            @pl.when(j >= NBUF)
            def _drain():
                wait_store(j - NBUF, sj)

            start_gather(j, sj)

        wait_gather(i, s)

        p0 = pos_v[i, pl.ds(0, 16)]
        p1 = pos_v[i, pl.ds(16, 16)]
        p2 = pos_v[i, pl.ds(32, 16)]
        p3 = pos_v[i, pl.ds(48, 16)]

        def add_row(r):
            plsc.addupdate(buf.at[s, r, pl.ds(0, 16)], p0)
            plsc.addupdate(buf.at[s, r, pl.ds(16, 16)], p1)
            plsc.addupdate(buf.at[s, r, pl.ds(32, 16)], p2)
            plsc.addupdate(buf.at[s, r, pl.ds(48, 16)], p3)

        plsc.parallel_loop(0, BC, 1, unroll=8)(add_row)
        start_store(i, s)
        return carry

    lax.fori_loop(0, S, body, 0)

    for i in range(S - NBUF, S):
        wait_store(i, i % NBUF)


@jax.jit
def _run(idx_t, token_table, pos_table):
    kern = pl.kernel(
        _embedder,
        out_type=jax.ShapeDtypeStruct((S, B, D), jnp.float32),
        mesh=plsc.VectorSubcoreMesh(core_axis_name="c", subcore_axis_name="s"),
        scratch_types=[
            pltpu.VMEM((S, BC), jnp.int32),
            pltpu.VMEM((S, D), jnp.float32),
            pltpu.VMEM((NBUF, BC, D), jnp.float32),
            pltpu.SemaphoreType.DMA((NBUF,)),
            pltpu.SemaphoreType.DMA((NBUF,)),
        ],
        compiler_params=pltpu.CompilerParams(use_tc_tiling_on_sc=False),
    )
    out = kern(idx_t, token_table, pos_table)
    return out.transpose(1, 0, 2)


def kernel(input, token_table, pos_table):
    return _run(input.T.astype(jnp.int32), token_table, pos_table)
